# baseline (speedup 1.0000x reference)
import functools
import math

import jax
from jax import lax
import jax.numpy as jnp
import numpy as np
from jax.experimental import pallas as pl
from jax.experimental.pallas import tpu as pltpu
from jax.experimental.pallas import tpu_sc as plsc

B, N, K = 8, 2048, 8
MUL0, MUL1 = 128, 64
D = MUL0 + 3 * MUL1
FW = 512
RB = 256
BIG = 1e30
E = B * N * K
HI = jax.lax.Precision.HIGHEST

NC, NS = 2, 16
NW = NC * NS
EPW = E // NW
CH = 64
NPC = CH // K
NCH = EPW // CH
NPW = N * B // NW

_TILE3 = np.zeros((3, 3 * MUL1), dtype=np.float32)
_SUM3 = np.zeros((3 * MUL1, MUL1), dtype=np.float32)
_REP3 = np.zeros((MUL1, 3 * MUL1), dtype=np.float32)
for _u in range(MUL1):
    for _m in range(3):
        _TILE3[_m, _u * 3 + _m] = 1.0
        _SUM3[_u * 3 + _m, _u] = 1.0
        _REP3[_u, _u * 3 + _m] = 1.0


def _tc1_kernel(feats_ref, coords_ref, wav_ref, tile3_ref, sum3_ref,
                rep3_ref, f_out_ref, idx_out_ref):
    b = pl.program_id(0)
    s = feats_ref[0, :, 0:MUL0]
    v = feats_ref[0, :, MUL0:D]
    c = coords_ref[0]

    ctile = jnp.dot(c, tile3_ref[:, :], preferred_element_type=jnp.float32,
                    precision=HI)
    p = jnp.dot(v * ctile, sum3_ref[:, :],
                preferred_element_type=jnp.float32, precision=HI)
    t = jnp.dot(s, wav_ref[:, :], preferred_element_type=jnp.float32,
                precision=HI)
    trep = jnp.dot(t, rep3_ref[:, :], preferred_element_type=jnp.float32,
                   precision=HI)
    f_out_ref[:, :] = jnp.concatenate(
        [t, trep * ctile, p, v], axis=1)

    sq = jnp.sum(c * c, axis=1, keepdims=True)

    for blk in range(N // RB):
        r0 = blk * RB
        cb = c[r0:r0 + RB]
        sqb = sq[r0:r0 + RB]
        g = jax.lax.dot_general(
            cb, c, (((1,), (1,)), ((), ())),
            preferred_element_type=jnp.float32)
        d2 = sqb + sq[:, 0][None, :] - 2.0 * g
        rows = r0 + jax.lax.broadcasted_iota(jnp.int32, (RB, N), 0)
        cols = jax.lax.broadcasted_iota(jnp.int32, (RB, N), 1)
        d2 = jnp.where(rows == cols, BIG, d2)
        work = d2
        iota_n = jax.lax.broadcasted_iota(jnp.int32, (N, 2), 0)
        col01 = jax.lax.broadcasted_iota(jnp.int32, (N, 2), 1)
        iota2 = jnp.where(col01 == 0, iota_n // 64,
                          iota_n % 64).astype(jnp.float32)
        for k in range(K):
            rowmin = jnp.min(work, axis=1, keepdims=True)
            eqf = (work <= rowmin).astype(jnp.float32)
            js2 = jnp.dot(eqf, iota2,
                          preferred_element_type=jnp.float32)
            jstar = (64.0 * js2[:, 0:1] + js2[:, 1:2]).astype(jnp.int32)
            idx_out_ref[r0:r0 + RB, k:k + 1] = jstar + b * N
            if k < K - 1:
                work = jnp.where(eqf != 0.0, BIG, work)


def _tc2_kernel(agg_ref, coords_ref, wcb_ref, tile3_ref, sum3_ref,
                rep3_ref, out_ref):
    agg = agg_ref[:, :]
    c = coords_ref[0]
    ctile = jnp.dot(c, tile3_ref[:, :], preferred_element_type=jnp.float32,
                    precision=HI)
    aggT = agg[:, 0:64]
    aggQ = agg[:, 64:256]
    aggP = agg[:, 256:320]
    aggV = agg[:, 320:512]
    dots = aggP - jnp.dot(aggV * ctile, sum3_ref[:, :],
                          preferred_element_type=jnp.float32,
                          precision=HI)
    out_s = jnp.dot(dots, wcb_ref[:, :], preferred_element_type=jnp.float32,
                    precision=HI)
    trep = jnp.dot(aggT, rep3_ref[:, :], preferred_element_type=jnp.float32,
                   precision=HI)
    out_ref[0, :, 0:MUL0] = out_s
    out_ref[0, :, MUL0:D] = aggQ - trep * ctile


def _sc_gather_sum(f_flat, idx_flat, nb):
    mesh = plsc.VectorSubcoreMesh(core_axis_name="c", subcore_axis_name="s")
    epw = nb * N * K // NW
    nch = epw // CH
    npw = nb * N // NW

    @functools.partial(
        pl.kernel, mesh=mesh,
        out_type=jax.ShapeDtypeStruct((nb * N, FW), jnp.float32),
        scratch_types=[
            pltpu.VMEM((epw,), jnp.int32),
            pltpu.VMEM((CH, FW), jnp.float32),
            pltpu.VMEM((CH, FW), jnp.float32),
            pltpu.VMEM((NPC, FW), jnp.float32),
            pltpu.VMEM((NPC, FW), jnp.float32),
            pltpu.SemaphoreType.DMA,
            pltpu.SemaphoreType.DMA,
            pltpu.SemaphoreType.DMA,
            pltpu.SemaphoreType.DMA,
        ],
    )
    def sc_kernel(f_hbm, i_hbm, o_hbm, idx_v, buf0, buf1, acc0, acc1,
                  gsem0, gsem1, osem0, osem1):
        wid = lax.axis_index("s") * NC + lax.axis_index("c")
        ebase = wid * epw
        nbase = wid * npw
        pltpu.async_copy(i_hbm.at[pl.ds(ebase, epw)], idx_v, gsem0).wait()

        def gather_start(g, buf, sem):
            pltpu.async_copy(f_hbm.at[idx_v.at[pl.ds(g * CH, CH)]], buf, sem)

        def gather_wait(g, buf, sem):
            pltpu.make_async_copy(
                f_hbm.at[idx_v.at[pl.ds(g * CH, CH)]], buf, sem).wait()

        def reduce_chunk(buf, acc):
            @pl.loop(0, NPC)
            def _node(n):
                r = n * K

                @pl.loop(0, FW, step=16)
                def _lane(cc):
                    sl = pl.ds(cc, 16)
                    acc[n, sl] = (
                        ((buf[r, sl] + buf[r + 1, sl])
                         + (buf[r + 2, sl] + buf[r + 3, sl]))
                        + ((buf[r + 4, sl] + buf[r + 5, sl])
                           + (buf[r + 6, sl] + buf[r + 7, sl])))

        def out_start(g, acc, sem):
            pltpu.async_copy(acc, o_hbm.at[pl.ds(nbase + g * NPC, NPC)], sem)

        def out_wait(g, acc, sem):
            pltpu.make_async_copy(
                acc, o_hbm.at[pl.ds(nbase + g * NPC, NPC)], sem).wait()

        gather_start(0, buf0, gsem0)

        @pl.loop(0, nch, step=2)
        def _pair(i):
            gather_start(i + 1, buf1, gsem1)
            gather_wait(i, buf0, gsem0)

            @pl.when(i > 0)
            def _():
                out_wait(i - 2, acc0, osem0)

            reduce_chunk(buf0, acc0)
            out_start(i, acc0, osem0)

            @pl.when(i + 2 < nch)
            def _():
                gather_start(i + 2, buf0, gsem0)

            gather_wait(i + 1, buf1, gsem1)

            @pl.when(i > 0)
            def _():
                out_wait(i - 1, acc1, osem1)

            reduce_chunk(buf1, acc1)
            out_start(i + 1, acc1, osem1)

        out_wait(nch - 2, acc0, osem0)
        out_wait(nch - 1, acc1, osem1)

    return sc_kernel(f_flat, idx_flat)


def _tc1_call(nf, co, wav, tile3, sum3, rep3, nb, h0):
    return pl.pallas_call(
        _tc1_kernel,
        grid=(nb,),
        in_specs=[
            pl.BlockSpec((1, N, D), lambda b: (h0 + b, 0, 0)),
            pl.BlockSpec((1, N, 3), lambda b: (h0 + b, 0, 0)),
            pl.BlockSpec((MUL0, MUL1), lambda b: (0, 0)),
            pl.BlockSpec((3, 3 * MUL1), lambda b: (0, 0)),
            pl.BlockSpec((3 * MUL1, MUL1), lambda b: (0, 0)),
            pl.BlockSpec((MUL1, 3 * MUL1), lambda b: (0, 0)),
        ],
        out_specs=[
            pl.BlockSpec((N, FW), lambda b: (b, 0)),
            pl.BlockSpec((N, K), lambda b: (b, 0)),
        ],
        out_shape=[
            jax.ShapeDtypeStruct((nb * N, FW), jnp.float32),
            jax.ShapeDtypeStruct((nb * N, K), jnp.int32),
        ],
    )(nf, co, wav, tile3, sum3, rep3)


def _tc2_call(agg, co, wcb, tile3, sum3, rep3, nb, h0):
    return pl.pallas_call(
        _tc2_kernel,
        grid=(nb,),
        in_specs=[
            pl.BlockSpec((N, FW), lambda b: (b, 0)),
            pl.BlockSpec((1, N, 3), lambda b: (h0 + b, 0, 0)),
            pl.BlockSpec((MUL1, MUL0), lambda b: (0, 0)),
            pl.BlockSpec((3, 3 * MUL1), lambda b: (0, 0)),
            pl.BlockSpec((3 * MUL1, MUL1), lambda b: (0, 0)),
            pl.BlockSpec((MUL1, 3 * MUL1), lambda b: (0, 0)),
        ],
        out_specs=pl.BlockSpec((1, N, D), lambda b: (b, 0, 0)),
        out_shape=jax.ShapeDtypeStruct((nb, N, D), jnp.float32),
    )(agg, co, wcb, tile3, sum3, rep3)


@jax.jit
def kernel(node_feats, coords, W_tp_a, W_tp_b, W_lin_s, W_lin_v):
    c_a = 1.0 / math.sqrt(MUL0)
    c_b = 1.0 / math.sqrt(MUL1 * 3)
    wav = (c_a / math.sqrt(MUL1)) * (W_tp_a @ W_lin_v)
    wcb = (c_b / math.sqrt(MUL0)) * (W_tp_b @ W_lin_s)
    tile3 = jnp.asarray(_TILE3)
    sum3 = jnp.asarray(_SUM3)
    rep3 = jnp.asarray(_REP3)

    nsplit = 2
    nb = B // nsplit
    outs = []
    stages = []
    for h in range(nsplit):
        f_tab, idx = _tc1_call(node_feats, coords, wav, tile3, sum3, rep3,
                               nb, h * nb)
        stages.append((f_tab, idx))
    for h in range(nsplit):
        f_tab, idx = stages[h]
        agg = _sc_gather_sum(f_tab, idx.reshape(nb * N * K), nb)
        outs.append(_tc2_call(agg, coords, wcb, tile3, sum3, rep3, nb,
                              h * nb))
    return jnp.concatenate(outs, axis=0)


# speedup vs baseline: 1.0948x; 1.0948x over previous
import functools
import math

import jax
from jax import lax
import jax.numpy as jnp
import numpy as np
from jax.experimental import pallas as pl
from jax.experimental.pallas import tpu as pltpu
from jax.experimental.pallas import tpu_sc as plsc

B, N, K = 8, 2048, 8
MUL0, MUL1 = 128, 64
D = MUL0 + 3 * MUL1
FW = 512
RB = 256
BIG = 1e30
E = B * N * K
HI = jax.lax.Precision.HIGHEST

NC, NS = 2, 16
NW = NC * NS
EPW = E // NW
CH = 64
NPC = CH // K
NCH = EPW // CH
NPW = N * B // NW

_TILE3 = np.zeros((3, 3 * MUL1), dtype=np.float32)
_SUM3 = np.zeros((3 * MUL1, MUL1), dtype=np.float32)
_REP3 = np.zeros((MUL1, 3 * MUL1), dtype=np.float32)
for _u in range(MUL1):
    for _m in range(3):
        _TILE3[_m, _u * 3 + _m] = 1.0
        _SUM3[_u * 3 + _m, _u] = 1.0
        _REP3[_u, _u * 3 + _m] = 1.0


def _tc1_kernel(feats_ref, coords_ref, wav_ref, tile3_ref, sum3_ref,
                rep3_ref, f_out_ref, idx_out_ref):
    b = pl.program_id(0)
    s = feats_ref[0, :, 0:MUL0]
    v = feats_ref[0, :, MUL0:D]
    c = coords_ref[0]

    ctile = jnp.dot(c, tile3_ref[:, :], preferred_element_type=jnp.float32,
                    precision=HI)
    p = jnp.dot(v * ctile, sum3_ref[:, :],
                preferred_element_type=jnp.float32, precision=HI)
    t = jnp.dot(s, wav_ref[:, :], preferred_element_type=jnp.float32,
                precision=HI)
    trep = jnp.dot(t, rep3_ref[:, :], preferred_element_type=jnp.float32,
                   precision=HI)
    f_out_ref[:, :] = jnp.concatenate(
        [t, trep * ctile, p, v], axis=1)

    sq = jnp.sum(c * c, axis=1, keepdims=True)

    for blk in range(N // RB):
        r0 = blk * RB
        cb = c[r0:r0 + RB]
        sqb = sq[r0:r0 + RB]
        g = jax.lax.dot_general(
            cb, c, (((1,), (1,)), ((), ())),
            preferred_element_type=jnp.float32)
        d2 = sqb + sq[:, 0][None, :] - 2.0 * g
        rows = r0 + jax.lax.broadcasted_iota(jnp.int32, (RB, N), 0)
        cols = jax.lax.broadcasted_iota(jnp.int32, (RB, N), 1)
        d2 = jnp.where(rows == cols, BIG, d2)
        work = d2
        for k in range(K):
            rowmin = jnp.min(work, axis=1, keepdims=True)
            eq = work <= rowmin
            jstar = jnp.min(jnp.where(eq, cols, N), axis=1,
                            keepdims=True)
            idx_out_ref[r0:r0 + RB, k:k + 1] = jstar + b * N
            if k < K - 1:
                work = jnp.where(eq, BIG, work)


def _tc2_kernel(agg_ref, coords_ref, wcb_ref, tile3_ref, sum3_ref,
                rep3_ref, *rest, h0=0, aliased=False):
    out_ref = rest[-1]
    agg = agg_ref[:, :]
    c = coords_ref[0]
    ctile = jnp.dot(c, tile3_ref[:, :], preferred_element_type=jnp.float32,
                    precision=HI)
    aggT = agg[:, 0:64]
    aggQ = agg[:, 64:256]
    aggP = agg[:, 256:320]
    aggV = agg[:, 320:512]
    dots = aggP - jnp.dot(aggV * ctile, sum3_ref[:, :],
                          preferred_element_type=jnp.float32,
                          precision=HI)
    out_s = jnp.dot(dots, wcb_ref[:, :], preferred_element_type=jnp.float32,
                    precision=HI)
    trep = jnp.dot(aggT, rep3_ref[:, :], preferred_element_type=jnp.float32,
                   precision=HI)
    out_ref[0, :, 0:MUL0] = out_s
    out_ref[0, :, MUL0:D] = aggQ - trep * ctile


def _sc_gather_sum(f_flat, idx_flat, nb):
    mesh = plsc.VectorSubcoreMesh(core_axis_name="c", subcore_axis_name="s")
    epw = nb * N * K // NW
    nch = epw // CH
    npw = nb * N // NW

    @functools.partial(
        pl.kernel, mesh=mesh,
        out_type=jax.ShapeDtypeStruct((nb * N, FW), jnp.float32),
        scratch_types=[
            pltpu.VMEM((epw,), jnp.int32),
            pltpu.VMEM((CH, FW), jnp.float32),
            pltpu.VMEM((CH, FW), jnp.float32),
            pltpu.VMEM((NPC, FW), jnp.float32),
            pltpu.VMEM((NPC, FW), jnp.float32),
            pltpu.SemaphoreType.DMA,
            pltpu.SemaphoreType.DMA,
            pltpu.SemaphoreType.DMA,
            pltpu.SemaphoreType.DMA,
        ],
    )
    def sc_kernel(f_hbm, i_hbm, o_hbm, idx_v, buf0, buf1, acc0, acc1,
                  gsem0, gsem1, osem0, osem1):
        wid = lax.axis_index("s") * NC + lax.axis_index("c")
        ebase = wid * epw
        nbase = wid * npw
        pltpu.async_copy(i_hbm.at[pl.ds(ebase, epw)], idx_v, gsem0).wait()

        def gather_start(g, buf, sem):
            pltpu.async_copy(f_hbm.at[idx_v.at[pl.ds(g * CH, CH)]], buf, sem)

        def gather_wait(g, buf, sem):
            pltpu.make_async_copy(
                f_hbm.at[idx_v.at[pl.ds(g * CH, CH)]], buf, sem).wait()

        def reduce_chunk(buf, acc):
            @pl.loop(0, NPC)
            def _node(n):
                r = n * K

                @pl.loop(0, FW, step=16)
                def _lane(cc):
                    sl = pl.ds(cc, 16)
                    acc[n, sl] = (
                        ((buf[r, sl] + buf[r + 1, sl])
                         + (buf[r + 2, sl] + buf[r + 3, sl]))
                        + ((buf[r + 4, sl] + buf[r + 5, sl])
                           + (buf[r + 6, sl] + buf[r + 7, sl])))

        def out_start(g, acc, sem):
            pltpu.async_copy(acc, o_hbm.at[pl.ds(nbase + g * NPC, NPC)], sem)

        def out_wait(g, acc, sem):
            pltpu.make_async_copy(
                acc, o_hbm.at[pl.ds(nbase + g * NPC, NPC)], sem).wait()

        gather_start(0, buf0, gsem0)

        @pl.loop(0, nch, step=2)
        def _pair(i):
            gather_start(i + 1, buf1, gsem1)
            gather_wait(i, buf0, gsem0)

            @pl.when(i > 0)
            def _():
                out_wait(i - 2, acc0, osem0)

            reduce_chunk(buf0, acc0)
            out_start(i, acc0, osem0)

            @pl.when(i + 2 < nch)
            def _():
                gather_start(i + 2, buf0, gsem0)

            gather_wait(i + 1, buf1, gsem1)

            @pl.when(i > 0)
            def _():
                out_wait(i - 1, acc1, osem1)

            reduce_chunk(buf1, acc1)
            out_start(i + 1, acc1, osem1)

        out_wait(nch - 2, acc0, osem0)
        out_wait(nch - 1, acc1, osem1)

    return sc_kernel(f_flat, idx_flat)


def _tc1_call(nf, co, wav, tile3, sum3, rep3, nb, h0):
    return pl.pallas_call(
        _tc1_kernel,
        grid=(nb,),
        in_specs=[
            pl.BlockSpec((1, N, D), lambda b: (h0 + b, 0, 0)),
            pl.BlockSpec((1, N, 3), lambda b: (h0 + b, 0, 0)),
            pl.BlockSpec((MUL0, MUL1), lambda b: (0, 0)),
            pl.BlockSpec((3, 3 * MUL1), lambda b: (0, 0)),
            pl.BlockSpec((3 * MUL1, MUL1), lambda b: (0, 0)),
            pl.BlockSpec((MUL1, 3 * MUL1), lambda b: (0, 0)),
        ],
        out_specs=[
            pl.BlockSpec((N, FW), lambda b: (b, 0)),
            pl.BlockSpec((N, K), lambda b: (b, 0)),
        ],
        out_shape=[
            jax.ShapeDtypeStruct((nb * N, FW), jnp.float32),
            jax.ShapeDtypeStruct((nb * N, K), jnp.int32),
        ],
    )(nf, co, wav, tile3, sum3, rep3)


def _tc2_call(agg, co, wcb, tile3, sum3, rep3, nb, h0, prev_out):
    in_specs = [
        pl.BlockSpec((N, FW), lambda b: (b, 0)),
        pl.BlockSpec((1, N, 3), lambda b: (h0 + b, 0, 0)),
        pl.BlockSpec((MUL1, MUL0), lambda b: (0, 0)),
        pl.BlockSpec((3, 3 * MUL1), lambda b: (0, 0)),
        pl.BlockSpec((3 * MUL1, MUL1), lambda b: (0, 0)),
        pl.BlockSpec((MUL1, 3 * MUL1), lambda b: (0, 0)),
    ]
    args = [agg, co, wcb, tile3, sum3, rep3]
    aliases = {}
    if prev_out is not None:
        in_specs.append(pl.BlockSpec(memory_space=pl.ANY))
        args.append(prev_out)
        aliases = {6: 0}
    return pl.pallas_call(
        functools.partial(_tc2_kernel, h0=h0, aliased=prev_out is not None),
        grid=(nb,),
        in_specs=in_specs,
        out_specs=pl.BlockSpec((1, N, D), lambda b: (h0 + b, 0, 0)),
        out_shape=jax.ShapeDtypeStruct((B, N, D), jnp.float32),
        input_output_aliases=aliases,
    )(*args)


@jax.jit
def kernel(node_feats, coords, W_tp_a, W_tp_b, W_lin_s, W_lin_v):
    c_a = 1.0 / math.sqrt(MUL0)
    c_b = 1.0 / math.sqrt(MUL1 * 3)
    wav = (c_a / math.sqrt(MUL1)) * (W_tp_a @ W_lin_v)
    wcb = (c_b / math.sqrt(MUL0)) * (W_tp_b @ W_lin_s)
    tile3 = jnp.asarray(_TILE3)
    sum3 = jnp.asarray(_SUM3)
    rep3 = jnp.asarray(_REP3)

    nsplit = 2
    nb = B // nsplit
    stages = []
    for h in range(nsplit):
        f_tab, idx = _tc1_call(node_feats, coords, wav, tile3, sum3, rep3,
                               nb, h * nb)
        stages.append((f_tab, idx))
    out = None
    for h in range(nsplit):
        f_tab, idx = stages[h]
        agg = _sc_gather_sum(f_tab, idx.reshape(nb * N * K), nb)
        out = _tc2_call(agg, coords, wcb, tile3, sum3, rep3, nb, h * nb, out)
    return out


# speedup vs baseline: 1.1469x; 1.0476x over previous
import functools
import math

import jax
from jax import lax
import jax.numpy as jnp
import numpy as np
from jax.experimental import pallas as pl
from jax.experimental.pallas import tpu as pltpu
from jax.experimental.pallas import tpu_sc as plsc

B, N, K = 8, 2048, 8
MUL0, MUL1 = 128, 64
D = MUL0 + 3 * MUL1
FW = 512
RB = 256
BIG = 1e30
E = B * N * K
HI = jax.lax.Precision.HIGHEST

NC, NS = 2, 16
NW = NC * NS
EPW = E // NW
CH = 64
NPC = CH // K
NCH = EPW // CH
NPW = N * B // NW

_TILE3 = np.zeros((3, 3 * MUL1), dtype=np.float32)
_SUM3 = np.zeros((3 * MUL1, MUL1), dtype=np.float32)
_REP3 = np.zeros((MUL1, 3 * MUL1), dtype=np.float32)
for _u in range(MUL1):
    for _m in range(3):
        _TILE3[_m, _u * 3 + _m] = 1.0
        _SUM3[_u * 3 + _m, _u] = 1.0
        _REP3[_u, _u * 3 + _m] = 1.0


def _tc1_kernel(feats_ref, coords_ref, wav_ref, tile3_ref, sum3_ref,
                rep3_ref, f_out_ref, idx_out_ref):
    b = pl.program_id(0)
    s = feats_ref[0, :, 0:MUL0]
    v = feats_ref[0, :, MUL0:D]
    c = coords_ref[0]

    ctile = jnp.dot(c, tile3_ref[:, :], preferred_element_type=jnp.float32,
                    precision=HI)
    p = jnp.dot(v * ctile, sum3_ref[:, :],
                preferred_element_type=jnp.float32, precision=HI)
    t = jnp.dot(s, wav_ref[:, :], preferred_element_type=jnp.float32,
                precision=HI)
    trep = jnp.dot(t, rep3_ref[:, :], preferred_element_type=jnp.float32,
                   precision=HI)
    f_out_ref[:, :] = jnp.concatenate(
        [t, trep * ctile, p, v], axis=1)

    sq = jnp.sum(c * c, axis=1, keepdims=True)

    for blk in range(N // RB):
        r0 = blk * RB
        cb = c[r0:r0 + RB]
        sqb = sq[r0:r0 + RB]
        g = jax.lax.dot_general(
            cb, c, (((1,), (1,)), ((), ())),
            preferred_element_type=jnp.float32)
        d2 = sqb + sq[:, 0][None, :] - 2.0 * g
        rows = r0 + jax.lax.broadcasted_iota(jnp.int32, (RB, N), 0)
        cols = jax.lax.broadcasted_iota(jnp.int32, (RB, N), 1)
        d2 = jnp.where(rows == cols, BIG, d2)
        work = d2
        for k in range(K):
            rowmin = jnp.min(work, axis=1, keepdims=True)
            eq = work <= rowmin
            jstar = jnp.min(jnp.where(eq, cols, N), axis=1,
                            keepdims=True)
            idx_out_ref[r0:r0 + RB, k:k + 1] = jstar + b * N
            if k < K - 1:
                work = jnp.where(eq, BIG, work)


def _tc2_kernel(agg_ref, coords_ref, wcb_ref, tile3_ref, sum3_ref,
                rep3_ref, *rest, h0=0, aliased=False):
    out_ref = rest[-1]
    agg = agg_ref[:, :]
    c = coords_ref[0]
    ctile = jnp.dot(c, tile3_ref[:, :], preferred_element_type=jnp.float32,
                    precision=HI)
    aggT = agg[:, 0:64]
    aggQ = agg[:, 64:256]
    aggP = agg[:, 256:320]
    aggV = agg[:, 320:512]
    dots = aggP - jnp.dot(aggV * ctile, sum3_ref[:, :],
                          preferred_element_type=jnp.float32,
                          precision=HI)
    out_s = jnp.dot(dots, wcb_ref[:, :], preferred_element_type=jnp.float32,
                    precision=HI)
    trep = jnp.dot(aggT, rep3_ref[:, :], preferred_element_type=jnp.float32,
                   precision=HI)
    out_ref[0, :, 0:MUL0] = out_s
    out_ref[0, :, MUL0:D] = aggQ - trep * ctile


def _sc_gather_sum(f_flat, idx_flat, nb):
    mesh = plsc.VectorSubcoreMesh(core_axis_name="c", subcore_axis_name="s")
    epw = nb * N * K // NW
    nch = epw // CH
    npw = nb * N // NW

    @functools.partial(
        pl.kernel, mesh=mesh,
        out_type=jax.ShapeDtypeStruct((nb * N, FW), jnp.float32),
        scratch_types=[
            pltpu.VMEM((epw,), jnp.int32),
            pltpu.VMEM((CH, FW), jnp.float32),
            pltpu.VMEM((CH, FW), jnp.float32),
            pltpu.VMEM((NPC, FW), jnp.float32),
            pltpu.VMEM((NPC, FW), jnp.float32),
            pltpu.SemaphoreType.DMA,
            pltpu.SemaphoreType.DMA,
            pltpu.SemaphoreType.DMA,
            pltpu.SemaphoreType.DMA,
        ],
    )
    def sc_kernel(f_hbm, i_hbm, o_hbm, idx_v, buf0, buf1, acc0, acc1,
                  gsem0, gsem1, osem0, osem1):
        wid = lax.axis_index("s") * NC + lax.axis_index("c")
        ebase = wid * epw
        nbase = wid * npw
        pltpu.async_copy(i_hbm.at[pl.ds(ebase, epw)], idx_v, gsem0).wait()

        def gather_start(g, buf, sem):
            pltpu.async_copy(f_hbm.at[idx_v.at[pl.ds(g * CH, CH)]], buf, sem)

        def gather_wait(g, buf, sem):
            pltpu.make_async_copy(
                f_hbm.at[idx_v.at[pl.ds(g * CH, CH)]], buf, sem).wait()

        def reduce_chunk(buf, acc):
            @pl.loop(0, NPC)
            def _node(n):
                r = n * K

                @pl.loop(0, FW, step=16)
                def _lane(cc):
                    sl = pl.ds(cc, 16)
                    acc[n, sl] = (
                        ((buf[r, sl] + buf[r + 1, sl])
                         + (buf[r + 2, sl] + buf[r + 3, sl]))
                        + ((buf[r + 4, sl] + buf[r + 5, sl])
                           + (buf[r + 6, sl] + buf[r + 7, sl])))

        def out_start(g, acc, sem):
            pltpu.async_copy(acc, o_hbm.at[pl.ds(nbase + g * NPC, NPC)], sem)

        def out_wait(g, acc, sem):
            pltpu.make_async_copy(
                acc, o_hbm.at[pl.ds(nbase + g * NPC, NPC)], sem).wait()

        gather_start(0, buf0, gsem0)

        @pl.loop(0, nch, step=2)
        def _pair(i):
            gather_start(i + 1, buf1, gsem1)
            gather_wait(i, buf0, gsem0)

            @pl.when(i > 0)
            def _():
                out_wait(i - 2, acc0, osem0)

            reduce_chunk(buf0, acc0)
            out_start(i, acc0, osem0)

            @pl.when(i + 2 < nch)
            def _():
                gather_start(i + 2, buf0, gsem0)

            gather_wait(i + 1, buf1, gsem1)

            @pl.when(i > 0)
            def _():
                out_wait(i - 1, acc1, osem1)

            reduce_chunk(buf1, acc1)
            out_start(i + 1, acc1, osem1)

        out_wait(nch - 2, acc0, osem0)
        out_wait(nch - 1, acc1, osem1)

    return sc_kernel(f_flat, idx_flat)


def _tc1_call(nf, co, wav, tile3, sum3, rep3, nb, h0):
    return pl.pallas_call(
        _tc1_kernel,
        grid=(nb,),
        in_specs=[
            pl.BlockSpec((1, N, D), lambda b: (h0 + b, 0, 0)),
            pl.BlockSpec((1, N, 3), lambda b: (h0 + b, 0, 0)),
            pl.BlockSpec((MUL0, MUL1), lambda b: (0, 0)),
            pl.BlockSpec((3, 3 * MUL1), lambda b: (0, 0)),
            pl.BlockSpec((3 * MUL1, MUL1), lambda b: (0, 0)),
            pl.BlockSpec((MUL1, 3 * MUL1), lambda b: (0, 0)),
        ],
        out_specs=[
            pl.BlockSpec((N, FW), lambda b: (b, 0)),
            pl.BlockSpec((N, K), lambda b: (b, 0)),
        ],
        out_shape=[
            jax.ShapeDtypeStruct((nb * N, FW), jnp.float32),
            jax.ShapeDtypeStruct((nb * N, K), jnp.int32),
        ],
    )(nf, co, wav, tile3, sum3, rep3)


def _tc2_call(agg, co, wcb, tile3, sum3, rep3, nb, h0, prev_out):
    in_specs = [
        pl.BlockSpec((N, FW), lambda b: (b, 0)),
        pl.BlockSpec((1, N, 3), lambda b: (h0 + b, 0, 0)),
        pl.BlockSpec((MUL1, MUL0), lambda b: (0, 0)),
        pl.BlockSpec((3, 3 * MUL1), lambda b: (0, 0)),
        pl.BlockSpec((3 * MUL1, MUL1), lambda b: (0, 0)),
        pl.BlockSpec((MUL1, 3 * MUL1), lambda b: (0, 0)),
    ]
    args = [agg, co, wcb, tile3, sum3, rep3]
    aliases = {}
    if prev_out is not None:
        in_specs.append(pl.BlockSpec(memory_space=pl.ANY))
        args.append(prev_out)
        aliases = {6: 0}
    return pl.pallas_call(
        functools.partial(_tc2_kernel, h0=h0, aliased=prev_out is not None),
        grid=(nb,),
        in_specs=in_specs,
        out_specs=pl.BlockSpec((1, N, D), lambda b: (h0 + b, 0, 0)),
        out_shape=jax.ShapeDtypeStruct((B, N, D), jnp.float32),
        input_output_aliases=aliases,
    )(*args)


@jax.jit
def kernel(node_feats, coords, W_tp_a, W_tp_b, W_lin_s, W_lin_v):
    c_a = 1.0 / math.sqrt(MUL0)
    c_b = 1.0 / math.sqrt(MUL1 * 3)
    wav = (c_a / math.sqrt(MUL1)) * (W_tp_a @ W_lin_v)
    wcb = (c_b / math.sqrt(MUL0)) * (W_tp_b @ W_lin_s)
    tile3 = jnp.asarray(_TILE3)
    sum3 = jnp.asarray(_SUM3)
    rep3 = jnp.asarray(_REP3)

    sizes = (5, 3)
    offs = (0, sizes[0])
    stages = []
    for h in range(2):
        f_tab, idx = _tc1_call(node_feats, coords, wav, tile3, sum3, rep3,
                               sizes[h], offs[h])
        stages.append((f_tab, idx))
    out = None
    for h in range(2):
        f_tab, idx = stages[h]
        agg = _sc_gather_sum(f_tab, idx.reshape(sizes[h] * N * K), sizes[h])
        out = _tc2_call(agg, coords, wcb, tile3, sum3, rep3, sizes[h],
                        offs[h], out)
    return out
